# baseline (speedup 1.0000x reference)
import numpy as np
import jax
import jax.numpy as jnp
from jax import lax
from jax.experimental import pallas as pl
from jax.experimental.pallas import tpu as pltpu
from jax.experimental.pallas import tpu_sc as plsc

_K = 15
_KP = 16
_A = 8
_KP_EXTENT = 1.2
_L = 16
_NC = 2
_NS = 16
_C = 64
_R = 4


def _mm_body(f_ref, w_ref, o_ref):
    o_ref[...] = jnp.dot(f_ref[...], w_ref[...],
                         preferred_element_type=jnp.float32
                         ).astype(jnp.bfloat16)


def _sc_body(g_h, px_h, py_h, pz_h, src_h, dst_h, kx_h, ky_h, kz_h, zero_h,
             out_h,
             px_v, py_v, pz_v, src_v, dst_v, kx_v, ky_v, kz_v,
             m_v, rows_v, msg_v, h_sh, sem, sem2):
    npad = h_sh.shape[0]
    ew = src_h.shape[0] // _NS
    c = lax.axis_index("c")
    s = lax.axis_index("s")
    base = s * ew
    pltpu.sync_copy(px_h, px_v)
    pltpu.sync_copy(py_h, py_v)
    pltpu.sync_copy(pz_h, pz_v)
    pltpu.sync_copy(kx_h, kx_v)
    pltpu.sync_copy(ky_h, ky_v)
    pltpu.sync_copy(kz_h, kz_v)
    rpt = npad // _NS
    pltpu.sync_copy(zero_h.at[pl.ds(s * rpt, rpt), :],
                    h_sh.at[pl.ds(s * rpt, rpt), :])
    plsc.subcore_barrier()
    m_v[pl.ds(_K * _L, _L)] = jnp.zeros((_L,), jnp.float32)

    inv_ext = jnp.float32(1.0 / _KP_EXTENT)
    ce = src_v.shape[0]
    nchunk = ew // ce
    nbc = ce // _L

    def _gather_copies(b, half, dsem):
        srcv = src_v[pl.ds(b * _L, _L)]
        grow = srcv * _KP + c
        return [pltpu.make_async_copy(
            g_h.at[grow + 2 * a],
            rows_v.at[half * _A + a], dsem)
            for a in range(_A)]

    def _issue(b, half, dsem):
        for cp in _gather_copies(b, half, dsem):
            cp.start()

    def _drain(b, half, dsem):
        for cp in _gather_copies(b, half, dsem):
            cp.wait()

    def _process(b, half):
        off = b * _L
        srcv = src_v[pl.ds(off, _L)]
        dstv = dst_v[pl.ds(off, _L)]
        yx = plsc.load_gather(px_v, [srcv]) - plsc.load_gather(px_v, [dstv])
        yy = plsc.load_gather(py_v, [srcv]) - plsc.load_gather(py_v, [dstv])
        yz = plsc.load_gather(pz_v, [srcv]) - plsc.load_gather(pz_v, [dstv])
        for k in range(_K):
            dx = yx - kx_v[pl.ds(k * _L, _L)]
            dy = yy - ky_v[pl.ds(k * _L, _L)]
            dz = yz - kz_v[pl.ds(k * _L, _L)]
            d2 = jnp.maximum(dx * dx + dy * dy + dz * dz, jnp.float32(1e-12))
            xi = jnp.int32(0x5F3759DF) - (plsc.bitcast(d2, jnp.int32) >> 1)
            x = plsc.bitcast(xi, jnp.float32)
            for _ in range(3):
                x = x * (jnp.float32(1.5) - jnp.float32(0.5) * d2 * x * x)
            dist = d2 * x
            m = jnp.maximum(jnp.float32(0.0),
                            jnp.float32(1.0) - dist * inv_ext)
            m_v[pl.ds(k * _L, _L)] = m

        @plsc.parallel_loop(0, _L, 1, unroll=2)
        def edge(j):
            accs = [jnp.zeros((_L,), jnp.float32) for _ in range(_R)]
            for a in range(_A):
                hb = half * _A + a
                c0 = plsc.load_gather(
                    m_v, [jnp.full((_L,), 2 * a * _L, jnp.int32) + j])
                c1 = plsc.load_gather(
                    m_v, [jnp.full((_L,), (2 * a + 1) * _L, jnp.int32) + j])
                for ss in range(4):
                    xw = rows_v[hb, j, pl.ds(ss * 2 * _L, 2 * _L)]
                    ea, eb = plsc.unpack(xw,
                                         format=plsc.PackFormat.INTERLEAVED)
                    coef = c0 if ss < 2 else c1
                    t = (ss % 2) * 2
                    accs[t] = accs[t] + coef * ea.astype(jnp.float32)
                    accs[t + 1] = accs[t + 1] + coef * eb.astype(jnp.float32)
            for r in range(_R):
                msg_v[j, pl.ds(r * _L, _L)] = accs[r]

        pltpu.sync_copy(msg_v, h_sh.at[dstv], add=True)

    def chunk(ch, carry):
        pltpu.sync_copy(src_h.at[pl.ds(base + ch * ce, ce)], src_v)
        pltpu.sync_copy(dst_h.at[pl.ds(base + ch * ce, ce)], dst_v)
        _issue(0, 0, sem)

        def pair(t, carry2):
            b0 = 2 * t
            b1 = b0 + 1
            _issue(b1, 1, sem2)
            _drain(b0, 0, sem)
            _process(b0, 0)

            @pl.when(b1 + 1 < nbc)
            def _():
                _issue(b1 + 1, 0, sem)

            _drain(b1, 1, sem2)
            _process(b1, 1)
            return carry2

        lax.fori_loop(0, nbc // 2, pair, 0)
        return carry

    lax.fori_loop(0, nchunk, chunk, 0)
    plsc.subcore_barrier()
    pltpu.sync_copy(h_sh.at[pl.ds(s * rpt, rpt), :],
                    out_h.at[pl.ds(c * npad + s * rpt, rpt), :])


def _sc_call(g2, px, py, pz, src, dst, kx, ky, kz, zero):
    n = px.shape[0]
    npad = zero.shape[0]
    e = src.shape[0]
    ew = e // _NS
    ce = 4000 if ew % 4000 == 0 else ew
    mesh = plsc.VectorSubcoreMesh(core_axis_name="c", subcore_axis_name="s",
                                  num_cores=_NC, num_subcores=_NS)
    f = pl.kernel(
        _sc_body,
        out_type=jax.ShapeDtypeStruct((_NC * npad, _C), jnp.float32),
        mesh=mesh,
        scratch_types=[
            pltpu.VMEM((n,), jnp.float32),
            pltpu.VMEM((n,), jnp.float32),
            pltpu.VMEM((n,), jnp.float32),
            pltpu.VMEM((ce,), jnp.int32),
            pltpu.VMEM((ce,), jnp.int32),
            pltpu.VMEM((_K * _L,), jnp.float32),
            pltpu.VMEM((_K * _L,), jnp.float32),
            pltpu.VMEM((_K * _L,), jnp.float32),
            pltpu.VMEM((_KP * _L,), jnp.float32),
            pltpu.VMEM((2 * _A, _L, 2 * _C), jnp.bfloat16),
            pltpu.VMEM((_L, _C), jnp.float32),
            pltpu.VMEM_SHARED((npad, _C), jnp.float32),
            pltpu.SemaphoreType.DMA,
            pltpu.SemaphoreType.DMA,
        ],
        compiler_params=pltpu.CompilerParams(needs_layout_passes=False,
                                             use_tc_tiling_on_sc=False),
    )
    return f(g2, px, py, pz, src, dst, kx, ky, kz, zero)


def kernel(feats, pos, edge_index, weights, kernel_points):
    n, in_dim = feats.shape
    kk, _, out_dim = weights.shape
    feats = feats.astype(jnp.float32)
    wt = weights.astype(jnp.float32).transpose(1, 0, 2)
    wt = jnp.concatenate(
        [wt, jnp.zeros((in_dim, _KP - kk, out_dim), jnp.float32)], axis=1)
    wt = wt.reshape(in_dim, _A, 2, 2, _C)
    fi = np.empty(_C, np.int32)
    for s2 in range(2):
        for l2 in range(_L):
            fi[32 * s2 + 2 * l2] = 32 * s2 + l2
            fi[32 * s2 + 2 * l2 + 1] = 32 * s2 + _L + l2
    wt = wt[..., fi]
    wc3 = wt.transpose(0, 1, 3, 2, 4).reshape(in_dim, _KP * 2 * _C)
    bm = 400 if n % 400 == 0 else 128
    g = pl.pallas_call(
        _mm_body,
        grid=(n // bm,),
        in_specs=[
            pl.BlockSpec((bm, in_dim), lambda i: (i, 0)),
            pl.BlockSpec((in_dim, _KP * 2 * _C), lambda i: (0, 0)),
        ],
        out_specs=pl.BlockSpec((bm, _KP * 2 * _C), lambda i: (i, 0)),
        out_shape=jax.ShapeDtypeStruct((n, _KP * 2 * _C), jnp.bfloat16),
    )(feats, wc3)
    g2 = g.reshape(n * _KP, 2 * _C)

    pos32 = pos.astype(jnp.float32)
    px, py, pz = pos32[:, 0], pos32[:, 1], pos32[:, 2]
    ei = edge_index.astype(jnp.int32)
    src, dst = ei[0], ei[1]
    kb = jnp.broadcast_to(
        kernel_points.astype(jnp.float32).T[:, :, None],
        (3, kk, _L)).reshape(3, kk * _L)
    npad = ((n + _NS * 8 - 1) // (_NS * 8)) * (_NS * 8)
    zero = jnp.zeros((npad, _C), jnp.float32)
    out = _sc_call(g2, px, py, pz, src, dst, kb[0], kb[1], kb[2], zero)
    return jnp.concatenate([out[:n], out[npad:npad + n]], axis=1)


# speedup vs baseline: 1.0676x; 1.0676x over previous
import numpy as np
import jax
import jax.numpy as jnp
from jax import lax
from jax.experimental import pallas as pl
from jax.experimental.pallas import tpu as pltpu
from jax.experimental.pallas import tpu_sc as plsc

_K = 15
_KP = 16
_A = 8
_KP_EXTENT = 1.2
_L = 16
_NC = 2
_NS = 16
_C = 64
_R = 4


def _mm_body(f_ref, w_ref, o_ref):
    o_ref[...] = jnp.dot(f_ref[...], w_ref[...],
                         preferred_element_type=jnp.float32
                         ).astype(jnp.bfloat16)


def _sc_body(g_h, px_h, py_h, pz_h, src_h, dst_h, kx_h, ky_h, kz_h, zero_h,
             out_h,
             px_v, py_v, pz_v, src_v, dst_v, kx_v, ky_v, kz_v,
             m_v, rows_v, msg_v, h_sh, sem, sem2):
    npad = h_sh.shape[0]
    ew = src_h.shape[0] // _NS
    c = lax.axis_index("c")
    s = lax.axis_index("s")
    base = s * ew
    pltpu.sync_copy(px_h, px_v)
    pltpu.sync_copy(py_h, py_v)
    pltpu.sync_copy(pz_h, pz_v)
    pltpu.sync_copy(kx_h, kx_v)
    pltpu.sync_copy(ky_h, ky_v)
    pltpu.sync_copy(kz_h, kz_v)
    rpt = npad // _NS
    pltpu.sync_copy(zero_h.at[pl.ds(s * rpt, rpt), :],
                    h_sh.at[pl.ds(s * rpt, rpt), :])
    plsc.subcore_barrier()
    m_v[pl.ds(_K * _L, _L)] = jnp.zeros((_L,), jnp.float32)

    inv_ext = jnp.float32(1.0 / _KP_EXTENT)
    ce = src_v.shape[0]
    nchunk = ew // ce
    nbc = ce // _L

    def _gather_copies(b, half, dsem):
        srcv = src_v[pl.ds(b * _L, _L)]
        grow = srcv * _KP + c
        return [pltpu.make_async_copy(
            g_h.at[grow + 2 * a],
            rows_v.at[half * _A + a], dsem)
            for a in range(_A)]

    def _issue(b, half, dsem):
        for cp in _gather_copies(b, half, dsem):
            cp.start()

    def _drain(b, half, dsem):
        for cp in _gather_copies(b, half, dsem):
            cp.wait()

    def _process(b, half):
        off = b * _L
        srcv = src_v[pl.ds(off, _L)]
        dstv = dst_v[pl.ds(off, _L)]
        yx = plsc.load_gather(px_v, [srcv]) - plsc.load_gather(px_v, [dstv])
        yy = plsc.load_gather(py_v, [srcv]) - plsc.load_gather(py_v, [dstv])
        yz = plsc.load_gather(pz_v, [srcv]) - plsc.load_gather(pz_v, [dstv])
        for k in range(_K):
            dx = yx - kx_v[pl.ds(k * _L, _L)]
            dy = yy - ky_v[pl.ds(k * _L, _L)]
            dz = yz - kz_v[pl.ds(k * _L, _L)]
            d2 = jnp.maximum(dx * dx + dy * dy + dz * dz, jnp.float32(1e-12))
            xi = jnp.int32(0x5F3759DF) - (plsc.bitcast(d2, jnp.int32) >> 1)
            x = plsc.bitcast(xi, jnp.float32)
            for _ in range(3):
                x = x * (jnp.float32(1.5) - jnp.float32(0.5) * d2 * x * x)
            dist = d2 * x
            m = jnp.maximum(jnp.float32(0.0),
                            jnp.float32(1.0) - dist * inv_ext)
            m_v[pl.ds(k * _L, _L)] = m

        @plsc.parallel_loop(0, _L, 1, unroll=4)
        def edge(j):
            accs = [jnp.zeros((_L,), jnp.float32) for _ in range(_R)]
            for a in range(_A):
                hb = half * _A + a
                c0 = plsc.load_gather(
                    m_v, [jnp.full((_L,), 2 * a * _L, jnp.int32) + j])
                c1 = plsc.load_gather(
                    m_v, [jnp.full((_L,), (2 * a + 1) * _L, jnp.int32) + j])
                for ss in range(4):
                    xw = rows_v[hb, j, pl.ds(ss * 2 * _L, 2 * _L)]
                    ea, eb = plsc.unpack(xw,
                                         format=plsc.PackFormat.INTERLEAVED)
                    coef = c0 if ss < 2 else c1
                    t = (ss % 2) * 2
                    accs[t] = accs[t] + coef * ea.astype(jnp.float32)
                    accs[t + 1] = accs[t + 1] + coef * eb.astype(jnp.float32)
            for r in range(_R):
                msg_v[j, pl.ds(r * _L, _L)] = accs[r]

        pltpu.sync_copy(msg_v, h_sh.at[dstv], add=True)

    def chunk(ch, carry):
        pltpu.sync_copy(src_h.at[pl.ds(base + ch * ce, ce)], src_v)
        pltpu.sync_copy(dst_h.at[pl.ds(base + ch * ce, ce)], dst_v)
        _issue(0, 0, sem)

        def pair(t, carry2):
            b0 = 2 * t
            b1 = b0 + 1
            _issue(b1, 1, sem2)
            _drain(b0, 0, sem)
            _process(b0, 0)

            @pl.when(b1 + 1 < nbc)
            def _():
                _issue(b1 + 1, 0, sem)

            _drain(b1, 1, sem2)
            _process(b1, 1)
            return carry2

        lax.fori_loop(0, nbc // 2, pair, 0)
        return carry

    lax.fori_loop(0, nchunk, chunk, 0)
    plsc.subcore_barrier()
    pltpu.sync_copy(h_sh.at[pl.ds(s * rpt, rpt), :],
                    out_h.at[pl.ds(c * npad + s * rpt, rpt), :])


def _sc_call(g2, px, py, pz, src, dst, kx, ky, kz, zero):
    n = px.shape[0]
    npad = zero.shape[0]
    e = src.shape[0]
    ew = e // _NS
    ce = 4000 if ew % 4000 == 0 else ew
    mesh = plsc.VectorSubcoreMesh(core_axis_name="c", subcore_axis_name="s",
                                  num_cores=_NC, num_subcores=_NS)
    f = pl.kernel(
        _sc_body,
        out_type=jax.ShapeDtypeStruct((_NC * npad, _C), jnp.float32),
        mesh=mesh,
        scratch_types=[
            pltpu.VMEM((n,), jnp.float32),
            pltpu.VMEM((n,), jnp.float32),
            pltpu.VMEM((n,), jnp.float32),
            pltpu.VMEM((ce,), jnp.int32),
            pltpu.VMEM((ce,), jnp.int32),
            pltpu.VMEM((_K * _L,), jnp.float32),
            pltpu.VMEM((_K * _L,), jnp.float32),
            pltpu.VMEM((_K * _L,), jnp.float32),
            pltpu.VMEM((_KP * _L,), jnp.float32),
            pltpu.VMEM((2 * _A, _L, 2 * _C), jnp.bfloat16),
            pltpu.VMEM((_L, _C), jnp.float32),
            pltpu.VMEM_SHARED((npad, _C), jnp.float32),
            pltpu.SemaphoreType.DMA,
            pltpu.SemaphoreType.DMA,
        ],
        compiler_params=pltpu.CompilerParams(needs_layout_passes=False,
                                             use_tc_tiling_on_sc=False),
    )
    return f(g2, px, py, pz, src, dst, kx, ky, kz, zero)


def kernel(feats, pos, edge_index, weights, kernel_points):
    n, in_dim = feats.shape
    kk, _, out_dim = weights.shape
    feats = feats.astype(jnp.float32)
    wt = weights.astype(jnp.float32).transpose(1, 0, 2)
    wt = jnp.concatenate(
        [wt, jnp.zeros((in_dim, _KP - kk, out_dim), jnp.float32)], axis=1)
    wt = wt.reshape(in_dim, _A, 2, 2, _C)
    fi = np.empty(_C, np.int32)
    for s2 in range(2):
        for l2 in range(_L):
            fi[32 * s2 + 2 * l2] = 32 * s2 + l2
            fi[32 * s2 + 2 * l2 + 1] = 32 * s2 + _L + l2
    wt = wt[..., fi]
    wc3 = wt.transpose(0, 1, 3, 2, 4).reshape(in_dim, _KP * 2 * _C)
    bm = 400 if n % 400 == 0 else 128
    g = pl.pallas_call(
        _mm_body,
        grid=(n // bm,),
        in_specs=[
            pl.BlockSpec((bm, in_dim), lambda i: (i, 0)),
            pl.BlockSpec((in_dim, _KP * 2 * _C), lambda i: (0, 0)),
        ],
        out_specs=pl.BlockSpec((bm, _KP * 2 * _C), lambda i: (i, 0)),
        out_shape=jax.ShapeDtypeStruct((n, _KP * 2 * _C), jnp.bfloat16),
    )(feats, wc3)
    g2 = g.reshape(n * _KP, 2 * _C)

    pos32 = pos.astype(jnp.float32)
    px, py, pz = pos32[:, 0], pos32[:, 1], pos32[:, 2]
    ei = edge_index.astype(jnp.int32)
    src, dst = ei[0], ei[1]
    kb = jnp.broadcast_to(
        kernel_points.astype(jnp.float32).T[:, :, None],
        (3, kk, _L)).reshape(3, kk * _L)
    npad = ((n + _NS * 8 - 1) // (_NS * 8)) * (_NS * 8)
    zero = jnp.zeros((npad, _C), jnp.float32)
    out = _sc_call(g2, px, py, pz, src, dst, kb[0], kb[1], kb[2], zero)
    return jnp.concatenate([out[:n], out[npad:npad + n]], axis=1)
